# baseline (speedup 1.0000x reference)
import functools
import math

import jax
import jax.numpy as jnp
from jax import lax
from jax.experimental import pallas as pl
from jax.experimental.pallas import tpu as pltpu
from jax.experimental.pallas import tpu_sc as plsc

_N = 100000
_B = 1024
_SMOOTHING = 0.1
_CONF = 1.0 - _SMOOTHING
_FILL = _SMOOTHING / (_N - 2)
_C1 = _FILL * (_N - 2) * math.log(_FILL) + _CONF * math.log(_CONF)

_NW = 32
_BPW = _B // _NW

_B_SC = 768
_B_TC = _B - _B_SC
_GPR = _B_SC // (_NW * 8)

_CW = 2048
_NCH = _N // _CW
_TAIL_OFF = _NCH * _CW
_TAIL_READ = 1792
_TAIL_FULL_ITERS = 13


def _sc_fetch_segments(x2d, t32):
    mesh = plsc.VectorSubcoreMesh(core_axis_name="c", subcore_axis_name="s")

    @functools.partial(
        pl.kernel,
        mesh=mesh,
        out_type=jax.ShapeDtypeStruct((_B, 128), jnp.float32),
        scratch_types=[
            pltpu.VMEM((_BPW,), jnp.int32),
            pltpu.VMEM((_BPW, 8, 128), jnp.float32),
            pltpu.VMEM((_BPW, 128), jnp.float32),
            pltpu.SemaphoreType.DMA,
        ],
    )
    def k(x_hbm, t_hbm, out_hbm, t_v, seg_v, stage_v, sem):
        wid = lax.axis_index("s") * 2 + lax.axis_index("c")
        base = wid * _BPW
        pltpu.sync_copy(t_hbm.at[pl.ds(base, _BPW)], t_v)
        tvecs = [t_v[pl.ds(16 * c, 16)] for c in range(_BPW // 16)]
        copies = []
        for j in range(_BPW):
            tj = tvecs[j // 16][j % 16]
            aj = pl.multiple_of((tj >> 7) << 7, 128)
            rj = pl.multiple_of(base + (j & ~7), 8)
            copies.append(pltpu.async_copy(
                x_hbm.at[pl.ds(rj, 8), pl.ds(aj, 128)], seg_v.at[j], sem))
        for c in copies:
            c.wait()
        for j in range(_BPW):
            for q in range(8):
                stage_v[j, pl.ds(16 * q, 16)] = (
                    seg_v[j, j % 8, pl.ds(16 * q, 16)])
        pltpu.sync_copy(stage_v, out_hbm.at[pl.ds(base, _BPW), :])

    return k(x2d, t32)


def _sc_partial_rowsums(x2d):
    mesh = plsc.VectorSubcoreMesh(core_axis_name="c", subcore_axis_name="s")

    @functools.partial(
        pl.kernel,
        mesh=mesh,
        out_type=jax.ShapeDtypeStruct((_B_SC, 128), jnp.float32),
        scratch_types=[
            pltpu.VMEM((8, _CW), jnp.float32),
            pltpu.VMEM((8, _CW), jnp.float32),
            pltpu.VMEM((8, 128), jnp.float32),
            pltpu.SemaphoreType.DMA,
            pltpu.SemaphoreType.DMA,
        ],
    )
    def k(x_hbm, out_hbm, buf0, buf1, stage_v, sem0, sem1):
        wid = lax.axis_index("s") * 2 + lax.axis_index("c")
        lanepos = lax.iota(jnp.int32, 16)
        zero16 = jnp.zeros((16,), jnp.float32)
        for g in range(_GPR):
            out_r0 = (wid * _GPR + g) * 8
            r0 = pl.multiple_of(_B_TC + out_r0, 8)

            def issue(c_idx, buf, sem):
                off = pl.multiple_of(c_idx * _CW, 128)
                return pltpu.async_copy(
                    x_hbm.at[pl.ds(r0, 8), pl.ds(off, _CW)], buf, sem)

            def drain(buf, sem):
                pltpu.make_async_copy(
                    x_hbm.at[pl.ds(r0, 8), pl.ds(0, _CW)], buf, sem).wait()

            def process(buf, accs):
                new = []
                for r in range(8):
                    def body(kk, a, _r=r, _buf=buf):
                        for u in range(8):
                            a = a + _buf[_r, pl.ds(kk * 128 + 16 * u, 16)]
                        return a
                    new.append(
                        lax.fori_loop(0, _CW // 128, body, accs[r]))
                return tuple(new)

            accs = tuple(zero16 for _ in range(8))
            cp0 = issue(0, buf0, sem0)
            cp1 = issue(1, buf1, sem1)
            cp0.wait()
            for r in range(8):
                row0 = buf0[r, pl.ds(0, 16)]
                buf0[r, pl.ds(0, 16)] = jnp.where(
                    lanepos == 0, 0.0, row0)
            accs = process(buf0, accs)
            issue(2, buf0, sem0)
            cp1.wait()
            accs = process(buf1, accs)
            issue(3, buf1, sem1)

            def loop_body(i, accs):
                drain(buf0, sem0)
                accs = process(buf0, accs)

                @pl.when(2 * i + 4 < _NCH)
                def _():
                    issue(2 * i + 4, buf0, sem0)
                drain(buf1, sem1)
                accs = process(buf1, accs)

                @pl.when(2 * i + 5 < _NCH)
                def _():
                    issue(2 * i + 5, buf1, sem1)
                return accs

            accs = lax.fori_loop(0, (_NCH - 2) // 2, loop_body, accs)

            tl = pl.multiple_of(_TAIL_OFF, 128)
            pltpu.async_copy(
                x_hbm.at[pl.ds(r0, 8), pl.ds(tl, _TAIL_READ)],
                buf0.at[:, pl.ds(0, _TAIL_READ)], sem0).wait()
            accs = list(accs)
            for r in range(8):
                def tbody(kk, a, _r=r):
                    for u in range(8):
                        a = a + buf0[_r, pl.ds(kk * 128 + 16 * u, 16)]
                    return a
                a = lax.fori_loop(0, _TAIL_FULL_ITERS, tbody, accs[r])
                a = a + buf0[r, pl.ds(1664, 16)] + buf0[r, pl.ds(1680, 16)]
                accs[r] = a
            for r in range(8):
                stage_v[r, pl.ds(0, 16)] = accs[r]
                for q in range(1, 8):
                    stage_v[r, pl.ds(16 * q, 16)] = zero16
            pltpu.sync_copy(stage_v, out_hbm.at[pl.ds(out_r0, 8), :])

    return k(x2d)


_BR = 16
_NS = 4
_NBLK = _B_TC // (_BR * _NS)


def _tc1_body(*refs):
    x_refs = refs[:_NS]
    v_ref, out_ref, acc_ref = refs[_NS:]
    i = pl.program_id(0)
    v = v_ref[...]
    part = jnp.float32(0.0)
    for k, x_ref in enumerate(x_refs):
        x = x_ref[...]
        rs = jnp.sum(x, axis=1, keepdims=True) - x[:, 0:1]
        part = part + jnp.sum(rs * v[k * _BR:(k + 1) * _BR, :])

    @pl.when(i == 0)
    def _():
        acc_ref[0, 0] = part

    @pl.when(i > 0)
    def _():
        acc_ref[0, 0] += part

    @pl.when(i == _NBLK - 1)
    def _():
        out_ref[0, 0] = acc_ref[0, 0]


def _make_x_spec(k):
    return pl.BlockSpec((_BR, _N), lambda i: (_NS * i + k, 0))


def _tc1_rowsum(x, vcol, interpret=False):
    return pl.pallas_call(
        _tc1_body,
        grid=(_NBLK,),
        in_specs=[_make_x_spec(k) for k in range(_NS)] + [
            pl.BlockSpec((_NS * _BR, 1), lambda i: (i, 0)),
        ],
        out_specs=pl.BlockSpec(memory_space=pltpu.SMEM),
        out_shape=jax.ShapeDtypeStruct((1, 1), jnp.float32),
        scratch_shapes=[pltpu.SMEM((1, 1), jnp.float32)],
        compiler_params=pltpu.CompilerParams(
            dimension_semantics=("arbitrary",)),
        interpret=interpret,
    )(*([x] * _NS), vcol)


def _tc2_body(s_ref, p_ref, seg_ref, tm_ref, v_ref, out_ref):
    v = v_ref[...]
    n_valid = jnp.sum(v)
    lane = lax.broadcasted_iota(jnp.int32, (_B, 128), 1)
    gmask = (lane == tm_ref[...]) & (v > 0.0)
    g_all = jnp.sum(jnp.where(gmask, seg_ref[...], 0.0))
    v_sc = v[_B_TC:, :]
    s_sc = jnp.sum(jnp.sum(p_ref[...], axis=1, keepdims=True) * v_sc)
    out_ref[0, 0] = (n_valid * _C1 - _FILL * (s_ref[0, 0] + s_sc)
                     + (_FILL - _CONF) * g_all) * (1.0 / _B)


def _tc2_combine(s_tc, psum, seg, tmod, vcol, interpret=False):
    return pl.pallas_call(
        _tc2_body,
        in_specs=[
            pl.BlockSpec(memory_space=pltpu.SMEM),
            pl.BlockSpec((_B_SC, 128), lambda: (0, 0)),
            pl.BlockSpec((_B, 128), lambda: (0, 0)),
            pl.BlockSpec((_B, 1), lambda: (0, 0)),
            pl.BlockSpec((_B, 1), lambda: (0, 0)),
        ],
        out_specs=pl.BlockSpec(memory_space=pltpu.SMEM),
        out_shape=jax.ShapeDtypeStruct((1, 1), jnp.float32),
        interpret=interpret,
    )(s_tc, psum, seg, tmod, vcol)


def kernel(log_inputs, targets):
    t32 = targets.astype(jnp.int32)
    seg = _sc_fetch_segments(log_inputs, t32)
    psum = _sc_partial_rowsums(log_inputs)
    tmod = (t32 & 127).reshape(_B, 1)
    vcol = (t32 != 0).astype(jnp.float32).reshape(_B, 1)
    s_tc = _tc1_rowsum(log_inputs, vcol)
    out = _tc2_combine(s_tc, psum, seg, tmod, vcol)
    return out.reshape(())


# speedup vs baseline: 1.1842x; 1.1842x over previous
import functools
import math

import jax
import jax.numpy as jnp
from jax import lax
from jax.experimental import pallas as pl
from jax.experimental.pallas import tpu as pltpu
from jax.experimental.pallas import tpu_sc as plsc

_N = 100000
_B = 1024
_SMOOTHING = 0.1
_CONF = 1.0 - _SMOOTHING
_FILL = _SMOOTHING / (_N - 2)
_C1 = _FILL * (_N - 2) * math.log(_FILL) + _CONF * math.log(_CONF)


_NW = 32
_BPW = _B // _NW


def _sc_fetch_segments(x2d, t32):
    mesh = plsc.VectorSubcoreMesh(core_axis_name="c", subcore_axis_name="s")

    @functools.partial(
        pl.kernel,
        mesh=mesh,
        out_type=jax.ShapeDtypeStruct((_B, 128), jnp.float32),
        scratch_types=[
            pltpu.VMEM((_BPW,), jnp.int32),
            pltpu.VMEM((_BPW, 8, 128), jnp.float32),
            pltpu.VMEM((_BPW, 128), jnp.float32),
            pltpu.SemaphoreType.DMA,
        ],
    )
    def k(x_hbm, t_hbm, out_hbm, t_v, seg_v, stage_v, sem):
        wid = lax.axis_index("s") * 2 + lax.axis_index("c")
        base = wid * _BPW
        pltpu.sync_copy(t_hbm.at[pl.ds(base, _BPW)], t_v)
        tvecs = [t_v[pl.ds(16 * c, 16)] for c in range(_BPW // 16)]
        copies = []
        for j in range(_BPW):
            tj = tvecs[j // 16][j % 16]
            aj = pl.multiple_of((tj >> 7) << 7, 128)
            rj = pl.multiple_of(base + (j & ~7), 8)
            copies.append(pltpu.async_copy(
                x_hbm.at[pl.ds(rj, 8), pl.ds(aj, 128)], seg_v.at[j], sem))
        for c in copies:
            c.wait()
        for j in range(_BPW):
            for q in range(8):
                stage_v[j, pl.ds(16 * q, 16)] = (
                    seg_v[j, j % 8, pl.ds(16 * q, 16)])
        pltpu.sync_copy(stage_v, out_hbm.at[pl.ds(base, _BPW), :])

    return k(x2d, t32)


_BR = 16
_NS = 4
_NBLK = _B // (_BR * _NS)


def _tc_body(*refs):
    x_refs = refs[:_NS]
    seg_ref, tm_ref, v_ref, out_ref, acc_ref = refs[_NS:]
    i = pl.program_id(0)
    v = v_ref[...]
    lane = lax.broadcasted_iota(jnp.int32, (_NS * _BR, 128), 1)
    gmask = (lane == tm_ref[...]) & (v > 0.0)
    g_part = jnp.sum(jnp.where(gmask, seg_ref[...], 0.0))
    part = _C1 * jnp.sum(v) + (_FILL - _CONF) * g_part
    for k, x_ref in enumerate(x_refs):
        x = x_ref[...]
        rs = jnp.sum(x, axis=1, keepdims=True) - x[:, 0:1]
        part = part - _FILL * jnp.sum(rs * v[k * _BR:(k + 1) * _BR, :])

    @pl.when(i == 0)
    def _():
        acc_ref[0, 0] = part

    @pl.when(i > 0)
    def _():
        acc_ref[0, 0] += part

    @pl.when(i == _NBLK - 1)
    def _():
        out_ref[0, 0] = acc_ref[0, 0] * (1.0 / _B)


def _make_x_spec(k):
    return pl.BlockSpec((_BR, _N), lambda i: (_NS * i + k, 0))


def _tc_reduce(x, seg, tmod, vcol, interpret=False):
    return pl.pallas_call(
        _tc_body,
        grid=(_NBLK,),
        in_specs=[_make_x_spec(k) for k in range(_NS)] + [
            pl.BlockSpec((_NS * _BR, 128), lambda i: (i, 0)),
            pl.BlockSpec((_NS * _BR, 1), lambda i: (i, 0)),
            pl.BlockSpec((_NS * _BR, 1), lambda i: (i, 0)),
        ],
        out_specs=pl.BlockSpec(memory_space=pltpu.SMEM),
        out_shape=jax.ShapeDtypeStruct((1, 1), jnp.float32),
        scratch_shapes=[pltpu.SMEM((1, 1), jnp.float32)],
        compiler_params=pltpu.CompilerParams(
            dimension_semantics=("arbitrary",)),
        interpret=interpret,
    )(*([x] * _NS), seg, tmod, vcol)


def kernel(log_inputs, targets):
    t32 = targets.astype(jnp.int32)
    seg = _sc_fetch_segments(log_inputs, t32)
    tmod = (t32 & 127).reshape(_B, 1)
    vcol = (t32 != 0).astype(jnp.float32).reshape(_B, 1)
    out = _tc_reduce(log_inputs, seg, tmod, vcol)
    return out.reshape(())
